# baseline (speedup 1.0000x reference)
import functools

import jax
import jax.numpy as jnp
from jax.experimental import pallas as pl

_N1 = 110000
_N2 = 100000
_D = 128
_H = 64
_C = 8
_R = 8
_B = 4
_E = 320000

_NBLK = 800
_EBLK = 2000


def _relu_root_body(x_ref, w_ref, xo_ref, ro_ref):
    x = jnp.maximum(x_ref[...], 0.0)
    xo_ref[...] = x
    ro_ref[...] = jnp.dot(x, w_ref[...], preferred_element_type=jnp.float32)


def _relu_root(x_pre, root):
    n, d = x_pre.shape
    k = root.shape[1]
    return pl.pallas_call(
        _relu_root_body,
        grid=(n // _NBLK,),
        in_specs=[
            pl.BlockSpec((_NBLK, d), lambda i: (i, 0)),
            pl.BlockSpec((d, k), lambda i: (0, 0)),
        ],
        out_specs=[
            pl.BlockSpec((_NBLK, d), lambda i: (i, 0)),
            pl.BlockSpec((_NBLK, k), lambda i: (i, 0)),
        ],
        out_shape=[
            jax.ShapeDtypeStruct((n, d), jnp.float32),
            jax.ShapeDtypeStruct((n, k), jnp.float32),
        ],
    )(x_pre, root)


def _edge_msg_body(nb, hw, x_ref, b_ref, w_ref, o_ref):
    hbe = jnp.dot(x_ref[...], b_ref[...], preferred_element_type=jnp.float32)
    w = w_ref[...]
    acc = w[:, 0:1] * hbe[:, 0:hw]
    for b in range(1, nb):
        acc = acc + w[:, b:b + 1] * hbe[:, b * hw:(b + 1) * hw]
    o_ref[...] = acc


def _edge_msg(x_src, bmat, w, hw):
    e, din = x_src.shape
    nb = w.shape[1]
    return pl.pallas_call(
        functools.partial(_edge_msg_body, nb, hw),
        grid=(e // _EBLK,),
        in_specs=[
            pl.BlockSpec((_EBLK, din), lambda i: (i, 0)),
            pl.BlockSpec((din, nb * hw), lambda i: (0, 0)),
            pl.BlockSpec((_EBLK, nb), lambda i: (i, 0)),
        ],
        out_specs=pl.BlockSpec((_EBLK, hw), lambda i: (i, 0)),
        out_shape=jax.ShapeDtypeStruct((e, hw), jnp.float32),
    )(x_src, bmat, w)


def _softmax_body(z_ref, o_ref):
    z = z_ref[...]
    z = z - jnp.max(z, axis=1, keepdims=True)
    ez = jnp.exp(z)
    o_ref[...] = ez / jnp.sum(ez, axis=1, keepdims=True)


def _softmax(z):
    n, c = z.shape
    return pl.pallas_call(
        _softmax_body,
        grid=(n // _NBLK,),
        in_specs=[pl.BlockSpec((_NBLK, c), lambda i: (i, 0))],
        out_specs=pl.BlockSpec((_NBLK, c), lambda i: (i, 0)),
        out_shape=jax.ShapeDtypeStruct((n, c), jnp.float32),
    )(z)


def kernel(edge_index_g2, edge_type_g2, edge_index_g1, all_node_embedding,
           basis1, comp1, root1, bias1, basis2, comp2, root2, bias2):
    src1 = edge_index_g1[0]
    dst1 = edge_index_g1[1]
    src2 = edge_index_g2[0]
    dst2 = edge_index_g2[1]
    et = edge_type_g2

    ones = jnp.ones((_E,), dtype=jnp.float32)

    gathered = jnp.take(all_node_embedding, src1, axis=0)
    agg0 = jax.ops.segment_sum(gathered, dst1, num_segments=_N1)
    deg = jax.ops.segment_sum(ones, dst1, num_segments=_N1)
    x_pre = agg0[:_N2] / jnp.maximum(deg[:_N2], 1.0)[:, None]

    keyid = dst2 * _R + et
    cnt = jax.ops.segment_sum(ones, keyid, num_segments=_N2 * _R)
    norm = 1.0 / jnp.maximum(jnp.take(cnt, keyid), 1.0)
    w1 = jnp.take(comp1, et, axis=0) * norm[:, None]
    w2 = jnp.take(comp2, et, axis=0) * norm[:, None]

    bmat1 = jnp.transpose(basis1, (1, 0, 2)).reshape(_D, _B * _H)
    x, rt1 = _relu_root(x_pre, root1)
    x_src = jnp.take(x, src2, axis=0)
    msg1 = _edge_msg(x_src, bmat1, w1, _H)
    agg1 = jax.ops.segment_sum(msg1, dst2, num_segments=_N2)
    pre2 = agg1 + rt1 + bias1[None, :]

    bmat2 = jnp.transpose(basis2, (1, 0, 2)).reshape(_H, _B * _C)
    h, rt2 = _relu_root(pre2, root2)
    h_src = jnp.take(h, src2, axis=0)
    msg2 = _edge_msg(h_src, bmat2, w2, _C)
    agg2 = jax.ops.segment_sum(msg2, dst2, num_segments=_N2)
    z = agg2 + rt2 + bias2[None, :]

    return _softmax(z)


# speedup vs baseline: 1.0351x; 1.0351x over previous
import functools

import jax
import jax.numpy as jnp
from jax.experimental import pallas as pl

_N1 = 110000
_N2 = 100000
_D = 128
_H = 64
_C = 8
_R = 8
_B = 4
_E = 320000

_NBLK = 800
_EBLK = 2000


def _relu_mm_body(x_ref, w_ref, o_ref):
    x = jnp.maximum(x_ref[...], 0.0)
    o_ref[...] = jnp.dot(x, w_ref[...], preferred_element_type=jnp.float32)


def _relu_mm(x, w):
    n, d = x.shape
    k = w.shape[1]
    return pl.pallas_call(
        _relu_mm_body,
        grid=(n // _NBLK,),
        in_specs=[
            pl.BlockSpec((_NBLK, d), lambda i: (i, 0)),
            pl.BlockSpec((d, k), lambda i: (0, 0)),
        ],
        out_specs=pl.BlockSpec((_NBLK, k), lambda i: (i, 0)),
        out_shape=jax.ShapeDtypeStruct((n, k), jnp.float32),
    )(x, w)


def _edge_combine_body(nb, hw, hb_ref, w_ref, o_ref):
    w = w_ref[...]
    acc = w[:, 0:1] * hb_ref[:, 0:hw]
    for b in range(1, nb):
        acc = acc + w[:, b:b + 1] * hb_ref[:, b * hw:(b + 1) * hw]
    o_ref[...] = acc


def _edge_combine(hb_src, w, hw):
    e = hb_src.shape[0]
    nb = w.shape[1]
    return pl.pallas_call(
        functools.partial(_edge_combine_body, nb, hw),
        grid=(e // _EBLK,),
        in_specs=[
            pl.BlockSpec((_EBLK, nb * hw), lambda i: (i, 0)),
            pl.BlockSpec((_EBLK, nb), lambda i: (i, 0)),
        ],
        out_specs=pl.BlockSpec((_EBLK, hw), lambda i: (i, 0)),
        out_shape=jax.ShapeDtypeStruct((e, hw), jnp.float32),
    )(hb_src, w)


def _softmax_body(z_ref, o_ref):
    z = z_ref[...]
    z = z - jnp.max(z, axis=1, keepdims=True)
    ez = jnp.exp(z)
    o_ref[...] = ez / jnp.sum(ez, axis=1, keepdims=True)


def _softmax(z):
    n, c = z.shape
    return pl.pallas_call(
        _softmax_body,
        grid=(n // _NBLK,),
        in_specs=[pl.BlockSpec((_NBLK, c), lambda i: (i, 0))],
        out_specs=pl.BlockSpec((_NBLK, c), lambda i: (i, 0)),
        out_shape=jax.ShapeDtypeStruct((n, c), jnp.float32),
    )(z)


def kernel(edge_index_g2, edge_type_g2, edge_index_g1, all_node_embedding,
           basis1, comp1, root1, bias1, basis2, comp2, root2, bias2):
    src1 = edge_index_g1[0]
    dst1 = edge_index_g1[1]
    src2 = edge_index_g2[0]
    dst2 = edge_index_g2[1]
    et = edge_type_g2

    ones = jnp.ones((_E,), dtype=jnp.float32)

    dst1c = jnp.minimum(dst1, _N2)
    gathered = jnp.take(all_node_embedding, src1, axis=0)
    agg0 = jax.ops.segment_sum(gathered, dst1c, num_segments=_N2 + 8)
    deg = jax.ops.segment_sum(ones, dst1c, num_segments=_N2 + 8)
    x_pre = agg0[:_N2] / jnp.maximum(deg[:_N2], 1.0)[:, None]

    keyid = dst2 * _R + et
    cnt = jax.ops.segment_sum(ones, keyid, num_segments=_N2 * _R)
    norm = 1.0 / jnp.maximum(jnp.take(cnt, keyid), 1.0)
    w1 = jnp.take(comp1, et, axis=0) * norm[:, None]
    w2 = jnp.take(comp2, et, axis=0) * norm[:, None]

    wmat1 = jnp.concatenate(
        [jnp.transpose(basis1, (1, 0, 2)).reshape(_D, _B * _H), root1], axis=1)
    wmat1 = jnp.pad(wmat1, ((0, 0), (0, 384 - _B * _H - _H)))
    hb1 = _relu_mm(x_pre, wmat1)
    rt1 = hb1[:, _B * _H:_B * _H + _H]
    hb1_src = jnp.take(hb1[:, :_B * _H], src2, axis=0)
    msg1 = _edge_combine(hb1_src, w1, _H)
    agg1 = jax.ops.segment_sum(msg1, dst2, num_segments=_N2)
    pre2 = agg1 + rt1 + bias1[None, :]

    wmat2 = jnp.concatenate(
        [jnp.transpose(basis2, (1, 0, 2)).reshape(_H, _B * _C), root2], axis=1)
    wmat2 = jnp.pad(wmat2, ((0, 0), (0, 128 - _B * _C - _C)))
    hb2 = _relu_mm(pre2, wmat2)
    rt2 = hb2[:, _B * _C:_B * _C + _C]
    hb2_src = jnp.take(hb2[:, :_B * _C], src2, axis=0)
    msg2 = _edge_combine(hb2_src, w2, _C)
    agg2 = jax.ops.segment_sum(msg2, dst2, num_segments=_N2)
    z = agg2 + rt2 + bias2[None, :]

    return _softmax(z)


# speedup vs baseline: 1.2264x; 1.1848x over previous
import functools

import jax
import jax.numpy as jnp
from jax import lax
from jax.experimental import pallas as pl
from jax.experimental.pallas import tpu as pltpu
from jax.experimental.pallas import tpu_sc as plsc

_N1 = 110000
_N2 = 100000
_D = 128
_H = 64
_C = 8
_R = 8
_B = 4
_E = 320000

_NBLK = 800
_EBLK = 2000


_SC_NC = 2
_SC_NS = 16
_SC_NW = _SC_NC * _SC_NS
_GCHUNK = 400


def _sc_gather_body(n_per_w, table_hbm, idx_hbm, out_hbm, idx_v, rows_v, sem):
    wid = lax.axis_index("s") * _SC_NC + lax.axis_index("c")
    base = wid * n_per_w

    def step(i, carry):
        off = base + i * _GCHUNK
        pltpu.sync_copy(idx_hbm.at[pl.ds(off, _GCHUNK)], idx_v)
        pltpu.async_copy(table_hbm.at[idx_v], rows_v, sem).wait()
        pltpu.sync_copy(rows_v, out_hbm.at[pl.ds(off, _GCHUNK)])
        return carry

    lax.fori_loop(0, n_per_w // _GCHUNK, step, 0)


def _sc_gather(table, idx):
    e = idx.shape[0]
    d = table.shape[1]
    n_per_w = e // _SC_NW
    mesh = plsc.VectorSubcoreMesh(core_axis_name="c", subcore_axis_name="s")
    body = functools.partial(_sc_gather_body, n_per_w)
    f = pl.kernel(
        body,
        mesh=mesh,
        out_type=jax.ShapeDtypeStruct((e, d), jnp.float32),
        scratch_types=[
            pltpu.VMEM((_GCHUNK,), jnp.int32),
            pltpu.VMEM((_GCHUNK, d), jnp.float32),
            pltpu.SemaphoreType.DMA,
        ],
    )
    return f(table, idx)


def _relu_mm_body(x_ref, w_ref, o_ref):
    x = jnp.maximum(x_ref[...], 0.0)
    o_ref[...] = jnp.dot(x, w_ref[...], preferred_element_type=jnp.float32)


def _relu_mm(x, w):
    n, d = x.shape
    k = w.shape[1]
    return pl.pallas_call(
        _relu_mm_body,
        grid=(n // _NBLK,),
        in_specs=[
            pl.BlockSpec((_NBLK, d), lambda i: (i, 0)),
            pl.BlockSpec((d, k), lambda i: (0, 0)),
        ],
        out_specs=pl.BlockSpec((_NBLK, k), lambda i: (i, 0)),
        out_shape=jax.ShapeDtypeStruct((n, k), jnp.float32),
    )(x, w)


def _edge_combine_body(nb, hw, hb_ref, w_ref, o_ref):
    w = w_ref[...]
    acc = w[:, 0:1] * hb_ref[:, 0:hw]
    for b in range(1, nb):
        acc = acc + w[:, b:b + 1] * hb_ref[:, b * hw:(b + 1) * hw]
    o_ref[...] = acc


def _edge_combine(hb_src, w, hw):
    e = hb_src.shape[0]
    nb = w.shape[1]
    return pl.pallas_call(
        functools.partial(_edge_combine_body, nb, hw),
        grid=(e // _EBLK,),
        in_specs=[
            pl.BlockSpec((_EBLK, nb * hw), lambda i: (i, 0)),
            pl.BlockSpec((_EBLK, nb), lambda i: (i, 0)),
        ],
        out_specs=pl.BlockSpec((_EBLK, hw), lambda i: (i, 0)),
        out_shape=jax.ShapeDtypeStruct((e, hw), jnp.float32),
    )(hb_src, w)


def _softmax_body(z_ref, o_ref):
    z = z_ref[...]
    z = z - jnp.max(z, axis=1, keepdims=True)
    ez = jnp.exp(z)
    o_ref[...] = ez / jnp.sum(ez, axis=1, keepdims=True)


def _softmax(z):
    n, c = z.shape
    return pl.pallas_call(
        _softmax_body,
        grid=(n // _NBLK,),
        in_specs=[pl.BlockSpec((_NBLK, c), lambda i: (i, 0))],
        out_specs=pl.BlockSpec((_NBLK, c), lambda i: (i, 0)),
        out_shape=jax.ShapeDtypeStruct((n, c), jnp.float32),
    )(z)


def kernel(edge_index_g2, edge_type_g2, edge_index_g1, all_node_embedding,
           basis1, comp1, root1, bias1, basis2, comp2, root2, bias2):
    src1 = edge_index_g1[0]
    dst1 = edge_index_g1[1]
    src2 = edge_index_g2[0]
    dst2 = edge_index_g2[1]
    et = edge_type_g2

    ones = jnp.ones((_E,), dtype=jnp.float32)

    dst1c = jnp.minimum(dst1, _N2)
    gathered = _sc_gather(all_node_embedding, src1)
    agg0 = jax.ops.segment_sum(gathered, dst1c, num_segments=_N2 + 8)
    deg = jax.ops.segment_sum(ones, dst1c, num_segments=_N2 + 8)
    x_pre = agg0[:_N2] / jnp.maximum(deg[:_N2], 1.0)[:, None]

    keyid = dst2 * _R + et
    cnt = jax.ops.segment_sum(ones, keyid, num_segments=_N2 * _R)
    norm = 1.0 / jnp.maximum(jnp.take(cnt, keyid), 1.0)
    w1 = jnp.take(comp1, et, axis=0) * norm[:, None]
    w2 = jnp.take(comp2, et, axis=0) * norm[:, None]

    wmat1 = jnp.concatenate(
        [jnp.transpose(basis1, (1, 0, 2)).reshape(_D, _B * _H), root1], axis=1)
    wmat1 = jnp.pad(wmat1, ((0, 0), (0, 384 - _B * _H - _H)))
    hb1 = _relu_mm(x_pre, wmat1)
    rt1 = hb1[:, _B * _H:_B * _H + _H]
    hb1_src = jnp.take(hb1[:, :_B * _H], src2, axis=0)
    msg1 = _edge_combine(hb1_src, w1, _H)
    agg1 = jax.ops.segment_sum(msg1, dst2, num_segments=_N2)
    pre2 = agg1 + rt1 + bias1[None, :]

    wmat2 = jnp.concatenate(
        [jnp.transpose(basis2, (1, 0, 2)).reshape(_H, _B * _C), root2], axis=1)
    wmat2 = jnp.pad(wmat2, ((0, 0), (0, 128 - _B * _C - _C)))
    hb2 = _relu_mm(pre2, wmat2)
    rt2 = hb2[:, _B * _C:_B * _C + _C]
    hb2_src = jnp.take(hb2[:, :_B * _C], src2, axis=0)
    msg2 = _edge_combine(hb2_src, w2, _C)
    agg2 = jax.ops.segment_sum(msg2, dst2, num_segments=_N2)
    z = agg2 + rt2 + bias2[None, :]

    return _softmax(z)
